# baseline (speedup 1.0000x reference)
import jax
import jax.numpy as jnp
from jax import lax
from jax.experimental import pallas as pl
from jax.experimental.pallas import tpu as pltpu
from jax.experimental.pallas import tpu_sc as plsc

B_ROWS = 4096
SEQ = 200
DIMS = 64

NC = 2
NS = 16
NW = NC * NS

TOTAL = B_ROWS * SEQ
PER_W = TOTAL // NW
CHUNK = 128
N_CHUNKS = PER_W // CHUNK


def _body(x_hbm, table_hbm, out_hbm, idx_v, rows_v, gsem):
    wid = lax.axis_index("s") * NC + lax.axis_index("c")
    pltpu.sync_copy(x_hbm.at[wid], idx_v)

    @pl.loop(0, N_CHUNKS)
    def _(g):
        pltpu.async_copy(table_hbm.at[idx_v.at[g]], rows_v, gsem).wait()
        pltpu.sync_copy(rows_v, out_hbm.at[wid, g])


_lookup = pl.kernel(
    _body,
    out_type=jax.ShapeDtypeStruct((NW, N_CHUNKS, CHUNK, DIMS), jnp.float32),
    mesh=plsc.VectorSubcoreMesh(core_axis_name="c", subcore_axis_name="s"),
    scratch_types=[
        pltpu.VMEM((N_CHUNKS, CHUNK), jnp.int32),
        pltpu.VMEM((CHUNK, DIMS), jnp.float32),
        pltpu.SemaphoreType.DMA,
    ],
    compiler_params=pltpu.CompilerParams(use_tc_tiling_on_sc=False),
)


@jax.jit
def kernel(x, table):
    x32 = x.astype(jnp.int32).reshape(NW, N_CHUNKS, CHUNK)
    out = _lookup(x32, table)
    return out.reshape(B_ROWS, SEQ, DIMS)


# speedup vs baseline: 1.1176x; 1.1176x over previous
import jax
import jax.numpy as jnp
from jax import lax
from jax.experimental import pallas as pl
from jax.experimental.pallas import tpu as pltpu
from jax.experimental.pallas import tpu_sc as plsc

B_ROWS = 4096
SEQ = 200
DIMS = 64

NC = 2
NS = 16
NW = NC * NS

TOTAL = B_ROWS * SEQ
PER_W = TOTAL // NW
CHUNK = 128
N_CHUNKS = PER_W // CHUNK


NBUF = 8
N_OUTER = N_CHUNKS // NBUF


def _body(x_hbm, table_hbm, out_hbm, idx_v, rows_v, gsems, osems):
    wid = lax.axis_index("s") * NC + lax.axis_index("c")
    pltpu.sync_copy(x_hbm.at[wid], idx_v)

    for b in range(NBUF):
        pltpu.async_copy(table_hbm.at[idx_v.at[b]], rows_v.at[b], gsems.at[b])

    @pl.loop(0, N_OUTER)
    def _(o):
        for b in range(NBUF):
            g = o * NBUF + b
            pltpu.make_async_copy(
                table_hbm.at[idx_v.at[g]], rows_v.at[b], gsems.at[b]
            ).wait()
            pltpu.async_copy(rows_v.at[b], out_hbm.at[wid, g], osems.at[b])

            @pl.when(o < N_OUTER - 1)
            def _():
                pltpu.make_async_copy(
                    rows_v.at[b], out_hbm.at[wid, g], osems.at[b]
                ).wait()
                pltpu.async_copy(
                    table_hbm.at[idx_v.at[g + NBUF]], rows_v.at[b], gsems.at[b]
                )

    for b in range(NBUF):
        pltpu.make_async_copy(
            rows_v.at[b], out_hbm.at[wid, N_CHUNKS - NBUF + b], osems.at[b]
        ).wait()


_lookup = pl.kernel(
    _body,
    out_type=jax.ShapeDtypeStruct((NW, N_CHUNKS, CHUNK, DIMS), jnp.float32),
    mesh=plsc.VectorSubcoreMesh(core_axis_name="c", subcore_axis_name="s"),
    scratch_types=[
        pltpu.VMEM((N_CHUNKS, CHUNK), jnp.int32),
        pltpu.VMEM((NBUF, CHUNK, DIMS), jnp.float32),
        pltpu.SemaphoreType.DMA((NBUF,)),
        pltpu.SemaphoreType.DMA((NBUF,)),
    ],
    compiler_params=pltpu.CompilerParams(use_tc_tiling_on_sc=False),
)


@jax.jit
def kernel(x, table):
    x32 = x.astype(jnp.int32).reshape(NW, N_CHUNKS, CHUNK)
    out = _lookup(x32, table)
    return out.reshape(B_ROWS, SEQ, DIMS)
